# baseline (speedup 1.0000x reference)
import math

import jax
import jax.numpy as jnp
from jax import lax
from jax.experimental import pallas as pl
from jax.experimental.pallas import tpu as pltpu
from jax.experimental.pallas import tpu_sc as plsc

N = 10000
E = 320000
D = 128
NUM_BRDFS = 4
NUM_BOUNCES = 8
FSM_GAMMA = 1e-3
SPEED_OF_SOUND = 343.0
MEAN_FREE_PATH = 5.0
AIR_ABS = 1e-3

WSCALE = math.exp(-AIR_ABS * MEAN_FREE_PATH) / 64.0
DECAY = math.exp(math.log(FSM_GAMMA) * (MEAN_FREE_PATH / SPEED_OF_SOUND))

NC = 2
NS = 16
L = 16
DH = D // NC
B = 128
NPAD = 10240
RPT = NPAD // NS
RCH = RPT // B
EPAD = 327680
EPT = EPAD // NS
NB = EPT // B
NROWS2 = NC * NPAD
NG = 4
NR = 8


def _sc_body(x_hbm, row_hbm, col_hbm, attr_hbm, coef_hbm,
             out_hbm, r_hbm, rcw_hbm,
             racc, gbufs, rbufs, abuf, coef_v, sg, ss, sr):
    gb = gbufs
    rb = rbufs
    c = lax.axis_index("c")
    s = lax.axis_index("s")
    rbase = c * NPAD + s * RPT
    lbase = s * RPT

    pltpu.sync_copy(coef_hbm, coef_v)
    cvec = coef_v[pl.ds(0, L)]
    c0 = cvec[0] * WSCALE
    c1 = cvec[1] * WSCALE
    c2 = cvec[2] * WSCALE
    c3 = cvec[3] * WSCALE
    coff = (c * NPAD).astype(jnp.int32)

    def init_batch(b, _):
        pltpu.sync_copy(row_hbm.at[s, b], rb[0].at[0])
        pltpu.sync_copy(col_hbm.at[s, b], rb[0].at[1])
        for k in range(NUM_BRDFS):
            pltpu.sync_copy(attr_hbm.at[pl.ds(k * EPAD + s * EPT + b * B, B)],
                            abuf.at[k])
        for h in range(B // L):
            sl = pl.ds(h * L, L)
            w = (abuf[0, sl] * c0 + abuf[1, sl] * c1
                 + abuf[2, sl] * c2 + abuf[3, sl] * c3)
            rb[0][2, sl] = plsc.bitcast(w, jnp.int32)
            rb[0][0, sl] = rb[0][0, sl] + coff
        pltpu.sync_copy(rb[0], rcw_hbm.at[c, s, b])
        return 0

    lax.fori_loop(0, NB, init_batch, 0, unroll=False)

    def init_rows(j, _):
        pltpu.sync_copy(x_hbm.at[pl.ds(rbase + j * B, B)], gb[0])
        pltpu.sync_copy(gb[0], r_hbm.at[pl.ds(rbase + j * B, B)])
        pltpu.sync_copy(gb[0], out_hbm.at[pl.ds(rbase + j * B, B)])

        def zz(e, _):
            for f in range(DH // L):
                gb[0][e, pl.ds(f * L, L)] = jnp.zeros((L,), jnp.float32)
            return 0

        lax.fori_loop(0, B, zz, 0, unroll=False)
        pltpu.sync_copy(gb[0], racc.at[pl.ds(lbase + j * B, B)])
        return 0

    lax.fori_loop(0, RCH, init_rows, 0, unroll=False)
    plsc.subcore_barrier()

    def bounce(t, _):
        for m in range(4):
            pltpu.async_copy(rcw_hbm.at[c, s, m], rb[m], sr[m])
        for m in range(2):
            pltpu.make_async_copy(rcw_hbm.at[c, s, 0], rb[m], sr[m]).wait()
            pltpu.async_copy(r_hbm.at[rb[m].at[0]], gb[m], sg[m])
        for m in range(2, 4):
            pltpu.async_copy(racc.at[pl.ds(0, B)], gb[m], ss[m])

        def oct_body(ho, _):
            for u in range(NR):
                j = ho * NR + u
                p = u % NG
                q = (u + 2) % NG
                m = u
                m4 = (u + 2) % NR
                m6 = (u + 4) % NR
                buf = gb[p]
                pltpu.make_async_copy(r_hbm.at[rb[m].at[0]], buf,
                                      sg[p]).wait()

                def mult_h(h, _):
                    wv = plsc.bitcast(rb[m][2, pl.ds(h * L, L)], jnp.float32)
                    for jj in range(L):
                        wj = jnp.full((L,), wv[jj], jnp.float32)
                        e = h * L + jj
                        for f in range(DH // L):
                            sl = pl.ds(f * L, L)
                            buf[e, sl] = buf[e, sl] * wj
                    return 0

                lax.fori_loop(0, B // L, mult_h, 0, unroll=False)
                pltpu.async_copy(buf, racc.at[rb[m].at[1]], ss[p], add=True)
                pltpu.make_async_copy(gb[q], racc.at[rb[m].at[1]],
                                      ss[q]).wait()
                pltpu.make_async_copy(rcw_hbm.at[c, s, 0], rb[m4], sr[m4]).wait()
                pltpu.async_copy(r_hbm.at[rb[m4].at[0]], gb[q], sg[q])
                jn = jnp.minimum(j + 4, NB - 1)
                pltpu.async_copy(rcw_hbm.at[c, s, jn], rb[m6], sr[m6])
            return 0

        lax.fori_loop(0, NB // NR, oct_body, 0, unroll=False)
        for m in range(2):
            pltpu.make_async_copy(r_hbm.at[rb[0].at[0]], gb[m], sg[m]).wait()
        for m in range(2, 4):
            pltpu.make_async_copy(gb[m], racc.at[rb[0].at[1]], ss[m]).wait()
        for m in range(2, 4):
            pltpu.make_async_copy(rcw_hbm.at[c, s, 0], rb[m], sr[m]).wait()
        plsc.subcore_barrier()

        def chunk_body(j, _):
            pltpu.sync_copy(racc.at[pl.ds(lbase + j * B, B)], gb[0])
            pltpu.sync_copy(out_hbm.at[pl.ds(rbase + j * B, B)], gb[1])

            def row_body(e, _):
                for f in range(DH // L):
                    sl = pl.ds(f * L, L)
                    v = gb[0][e, sl] * DECAY
                    gb[0][e, sl] = v
                    gb[1][e, sl] = gb[1][e, sl] + v
                return 0

            lax.fori_loop(0, B, row_body, 0, unroll=False)
            pltpu.sync_copy(gb[0], r_hbm.at[pl.ds(rbase + j * B, B)])
            pltpu.sync_copy(gb[1], out_hbm.at[pl.ds(rbase + j * B, B)])

            def zz(e, _):
                for f in range(DH // L):
                    gb[0][e, pl.ds(f * L, L)] = jnp.zeros((L,), jnp.float32)
                return 0

            lax.fori_loop(0, B, zz, 0, unroll=False)
            pltpu.sync_copy(gb[0], racc.at[pl.ds(lbase + j * B, B)])
            return 0

        lax.fori_loop(0, RCH, chunk_body, 0, unroll=False)
        plsc.subcore_barrier()
        return 0

    lax.fori_loop(0, NUM_BOUNCES, bounce, 0, unroll=False)


def kernel(x, edge_index, edge_attr, brdf_coeffs):
    x2 = x.reshape(N, NC, DH).transpose(1, 0, 2)
    x2 = jnp.pad(x2, ((0, 0), (0, NPAD - N), (0, 0))).reshape(NROWS2, DH)

    row = jnp.pad(edge_index[0].astype(jnp.int32), (0, EPAD - E))
    col = jnp.pad(edge_index[1].astype(jnp.int32), (0, EPAD - E))
    row2 = row.reshape(NS, NB, B)
    col2 = col.reshape(NS, NB, B)
    attr2 = jnp.pad(edge_attr.astype(jnp.float32).T,
                    ((0, 0), (0, EPAD - E))).reshape(NUM_BRDFS * EPAD)
    coef = jnp.pad(brdf_coeffs.astype(jnp.float32), (0, L - NUM_BRDFS))

    mesh = plsc.VectorSubcoreMesh(core_axis_name="c", subcore_axis_name="s",
                                  num_cores=NC, num_subcores=NS)
    f32 = jnp.float32
    i32 = jnp.int32
    run = pl.kernel(
        _sc_body,
        out_type=(jax.ShapeDtypeStruct((NROWS2, DH), f32),
                  jax.ShapeDtypeStruct((NROWS2, DH), f32),
                  jax.ShapeDtypeStruct((NC, NS, NB, 3, B), i32)),
        mesh=mesh,
        scratch_types=[
            pltpu.VMEM_SHARED((NPAD, DH), f32),
            [pltpu.VMEM((B, DH), f32) for _ in range(NG)],
            [pltpu.VMEM((3, B), i32) for _ in range(NR)],
            pltpu.VMEM((NUM_BRDFS, B), f32),
            pltpu.VMEM((L,), f32),
            [pltpu.SemaphoreType.DMA for _ in range(NG)],
            [pltpu.SemaphoreType.DMA for _ in range(NG)],
            [pltpu.SemaphoreType.DMA for _ in range(NR)],
        ],
        compiler_params=pltpu.CompilerParams(use_tc_tiling_on_sc=False,
                                             needs_layout_passes=False),
    )
    out2, _, _ = run(x2, row2, col2, attr2, coef)
    out = out2.reshape(NC, NPAD, DH)[:, :N]
    return out.transpose(1, 0, 2).reshape(N, D)
